# Initial kernel scaffold; baseline (speedup 1.0000x reference)
import jax
import jax.numpy as jnp
from jax.experimental import pallas as pl


def kernel(x, edge_index, W1, att_src1, att_dst1, bias1, W2, att_src2, att_dst2, bias2):
    raise NotImplementedError("write your pallas kernel here")


# speedup vs baseline: 40.4049x; 40.4049x over previous
import functools

import jax
import jax.numpy as jnp
from jax import lax
from jax.experimental import pallas as pl
from jax.experimental.pallas import tpu as pltpu
from jax.experimental.pallas import tpu_sc as plsc

N = 10000
E = 320000
F = 128
H1, C1 = 8, 8
D1 = H1 * C1
C2 = 4
L = 16

NC, NS = 2, 16
NW = NC * NS
EW = E // NW
B = 80
NCH = EW // B

_f32 = jnp.float32


def _lane_gather(x, idx):
    return lax.gather(
        x, idx[:, None],
        lax.GatherDimensionNumbers(offset_dims=(), collapsed_slice_dims=(0,),
                                   start_index_map=(0,)),
        (1,), mode=lax.GatherScatterMode.PROMISE_IN_BOUNDS)


def _mesh():
    return plsc.VectorSubcoreMesh(core_axis_name="c", subcore_axis_name="s",
                                  num_cores=NC, num_subcores=NS)


def _tc_proj_body(x_ref, w_ref, as_ref, ad_ref, xw_ref, a1_ref):
    xw = jnp.dot(x_ref[...], w_ref[...], preferred_element_type=_f32)
    xw_ref[...] = xw
    asrc = jnp.dot(xw, as_ref[...], preferred_element_type=_f32)
    adst = jnp.dot(xw, ad_ref[...], preferred_element_type=_f32)
    a1_ref[...] = jnp.concatenate([asrc, adst], axis=1)


def _tc_proj(x, W1, As, Ad):
    bn = 1000
    return pl.pallas_call(
        _tc_proj_body,
        grid=(N // bn,),
        in_specs=[pl.BlockSpec((bn, F), lambda i: (i, 0)),
                  pl.BlockSpec((F, D1), lambda i: (0, 0)),
                  pl.BlockSpec((D1, H1), lambda i: (0, 0)),
                  pl.BlockSpec((D1, H1), lambda i: (0, 0))],
        out_specs=[pl.BlockSpec((bn, D1), lambda i: (i, 0)),
                   pl.BlockSpec((bn, L), lambda i: (i, 0))],
        out_shape=[jax.ShapeDtypeStruct((N, D1), _f32),
                   jax.ShapeDtypeStruct((N, L), _f32)],
    )(x, W1, As, Ad)


def _tc_rdenom_body(d_ref, out_ref):
    out_ref[...] = 1.0 / (d_ref[0] + d_ref[1] + 1e-16)


def _tc_rdenom(denp):
    shp = denp.shape[1:]
    return pl.pallas_call(
        _tc_rdenom_body,
        out_shape=jax.ShapeDtypeStruct(shp, _f32),
    )(denp)


def _tc_layer2_body(op_ref, b1_ref, w2_ref, as2_ref, ad2_ref,
                    h_ref, xw2_ref, a2_ref):
    o = op_ref[0] + op_ref[1] + b1_ref[...]
    h = jnp.maximum(o, 0.0)
    h_ref[...] = h
    xw2 = jnp.dot(h, w2_ref[...], preferred_element_type=_f32)
    xw2_ref[...] = xw2
    asrc2 = jnp.dot(xw2, as2_ref[...], preferred_element_type=_f32)
    adst2 = jnp.dot(xw2, ad2_ref[...], preferred_element_type=_f32)
    a2_ref[...] = jnp.concatenate([asrc2, adst2], axis=1)


def _tc_layer2(outp, bias1, W2, as2, ad2):
    bn = 1000
    return pl.pallas_call(
        _tc_layer2_body,
        grid=(N // bn,),
        in_specs=[pl.BlockSpec((2, bn, D1), lambda i: (0, i, 0)),
                  pl.BlockSpec((1, D1), lambda i: (0, 0)),
                  pl.BlockSpec((D1, C2), lambda i: (0, 0)),
                  pl.BlockSpec((C2, 1), lambda i: (0, 0)),
                  pl.BlockSpec((C2, 1), lambda i: (0, 0))],
        out_specs=[pl.BlockSpec((bn, D1), lambda i: (i, 0)),
                   pl.BlockSpec((bn, C2), lambda i: (i, 0)),
                   pl.BlockSpec((bn, 2), lambda i: (i, 0))],
        out_shape=[jax.ShapeDtypeStruct((N, D1), _f32),
                   jax.ShapeDtypeStruct((N, C2), _f32),
                   jax.ShapeDtypeStruct((N, 2), _f32)],
    )(outp, bias1, W2, as2, ad2)


def _tc_final_body(op_ref, b2_ref, z_ref):
    z_ref[...] = op_ref[0, :, :C2] + op_ref[1, :, :C2] + b2_ref[...]


def _tc_final(out2p, bias2):
    return pl.pallas_call(
        _tc_final_body,
        out_shape=jax.ShapeDtypeStruct((N, C2), _f32),
    )(out2p, bias2)


def _sc_edge_softmax1(e_src, e_dst, a1, zer16):
    @functools.partial(
        pl.kernel,
        compiler_params=pltpu.CompilerParams(use_tc_tiling_on_sc=False, needs_layout_passes=False),
        out_type=(jax.ShapeDtypeStruct((E, L), _f32),
                  jax.ShapeDtypeStruct((NC, N, L), _f32)),
        mesh=_mesh(),
        scratch_types=[
            pltpu.VMEM((B,), jnp.int32), pltpu.VMEM((B,), jnp.int32),
            pltpu.VMEM((B, L), _f32), pltpu.VMEM((B, L), _f32),
            pltpu.VMEM((B, L), _f32),
            pltpu.VMEM_SHARED((N, L), _f32),
            pltpu.SemaphoreType.DMA, pltpu.SemaphoreType.DMA,
        ])
    def k(eis, eid, a1h, zr, ex_out, den_out,
          src_v, dst_v, srows, drows, exb, den_sh, sem1, sem2):
        cid = lax.axis_index("c")
        sid = lax.axis_index("s")
        wid = cid * NS + sid
        rot8 = (lax.iota(jnp.int32, L) % 8) + 8

        @pl.when(sid == 0)
        def _zero():
            pltpu.sync_copy(zr, den_sh)
        plsc.subcore_barrier()

        @pl.loop(0, NCH)
        def _chunk(i):
            base = wid * EW + i * B
            pltpu.sync_copy(eis.at[pl.ds(base, B)], src_v)
            pltpu.sync_copy(eid.at[pl.ds(base, B)], dst_v)
            ca = pltpu.async_copy(a1h.at[src_v], srows, sem1)
            cb = pltpu.async_copy(a1h.at[dst_v], drows, sem2)
            ca.wait()
            cb.wait()

            @pl.loop(0, B)
            def _edge(j):
                e = srows[j] + _lane_gather(drows[j], rot8)
                e = jnp.where(e >= 0.0, e, e * _f32(0.2))
                exb[j] = jnp.exp(e)

            pltpu.sync_copy(exb, ex_out.at[pl.ds(base, B)])
            pltpu.sync_copy(exb, den_sh.at[dst_v], add=True)

        plsc.subcore_barrier()

        @pl.when(sid == 0)
        def _flush():
            pltpu.sync_copy(den_sh, den_out.at[cid])

    return k(e_src, e_dst, a1, zer16)


def _sc_aggregate1(e_src, e_dst, xw, rdenom, ex, zer64):
    @functools.partial(
        pl.kernel,
        compiler_params=pltpu.CompilerParams(use_tc_tiling_on_sc=False, needs_layout_passes=False),
        out_type=jax.ShapeDtypeStruct((NC, N, D1), _f32),
        mesh=_mesh(),
        scratch_types=[
            pltpu.VMEM((B,), jnp.int32), pltpu.VMEM((B,), jnp.int32),
            pltpu.VMEM((B, D1), _f32), pltpu.VMEM((B, L), _f32),
            pltpu.VMEM((B, L), _f32), pltpu.VMEM((B, D1), _f32),
            pltpu.VMEM_SHARED((N, D1), _f32),
            pltpu.SemaphoreType.DMA, pltpu.SemaphoreType.DMA,
        ])
    def k(eis, eid, xwh, rdh, exh, zr, out_p,
          src_v, dst_v, xwrows, rdrows, exb, msgb, out_sh, sem1, sem2):
        cid = lax.axis_index("c")
        sid = lax.axis_index("s")
        wid = cid * NS + sid
        lane8 = lax.iota(jnp.int32, L) // 8

        @pl.when(sid == 0)
        def _zero():
            pltpu.sync_copy(zr, out_sh)
        plsc.subcore_barrier()

        @pl.loop(0, NCH)
        def _chunk(i):
            base = wid * EW + i * B
            pltpu.sync_copy(eis.at[pl.ds(base, B)], src_v)
            pltpu.sync_copy(eid.at[pl.ds(base, B)], dst_v)
            cx = pltpu.async_copy(xwh.at[src_v], xwrows, sem1)
            cr = pltpu.async_copy(rdh.at[dst_v], rdrows, sem2)
            pltpu.sync_copy(exh.at[pl.ds(base, B)], exb)
            cx.wait()
            cr.wait()

            @pl.loop(0, B)
            def _edge(j):
                alpha = exb[j] * rdrows[j]
                for kk in range(4):
                    a_k = _lane_gather(alpha, 2 * kk + lane8)
                    sl = pl.ds(kk * L, L)
                    msgb[j, sl] = xwrows[j, sl] * a_k

            pltpu.sync_copy(msgb, out_sh.at[dst_v], add=True)

        plsc.subcore_barrier()

        @pl.when(sid == 0)
        def _flush():
            pltpu.sync_copy(out_sh, out_p.at[cid])

    return k(e_src, e_dst, xw, rdenom, ex, zer64)


def _sc_edge_softmax2(e_src, e_dst, a2, zer1):
    @functools.partial(
        pl.kernel,
        compiler_params=pltpu.CompilerParams(use_tc_tiling_on_sc=False, needs_layout_passes=False),
        out_type=(jax.ShapeDtypeStruct((E,), _f32),
                  jax.ShapeDtypeStruct((NC, N), _f32)),
        mesh=_mesh(),
        scratch_types=[
            pltpu.VMEM((N, 2), _f32),
            pltpu.VMEM((B,), jnp.int32), pltpu.VMEM((B,), jnp.int32),
            pltpu.VMEM((B,), _f32),
            pltpu.VMEM_SHARED((N,), _f32),
        ])
    def k(eis, eid, a2h, zr, ex_out, den_out,
          a2_v, src_v, dst_v, exb, den_sh):
        cid = lax.axis_index("c")
        sid = lax.axis_index("s")
        wid = cid * NS + sid

        pltpu.sync_copy(a2h, a2_v)

        @pl.when(sid == 0)
        def _zero():
            pltpu.sync_copy(zr, den_sh)
        plsc.subcore_barrier()

        col0 = lax.iota(jnp.int32, L) * 0
        col1 = col0 + 1

        @pl.loop(0, NCH)
        def _chunk(i):
            base = wid * EW + i * B
            pltpu.sync_copy(eis.at[pl.ds(base, B)], src_v)
            pltpu.sync_copy(eid.at[pl.ds(base, B)], dst_v)

            @pl.loop(0, B // L)
            def _grp(g):
                sv = src_v[pl.ds(g * L, L)]
                dv = dst_v[pl.ds(g * L, L)]
                av = plsc.load_gather(a2_v, [sv, col0])
                bv = plsc.load_gather(a2_v, [dv, col1])
                e = av + bv
                e = jnp.where(e >= 0.0, e, e * _f32(0.2))
                exb[pl.ds(g * L, L)] = jnp.exp(e)

            pltpu.sync_copy(exb, ex_out.at[pl.ds(base, B)])
            pltpu.sync_copy(exb, den_sh.at[dst_v], add=True)

        plsc.subcore_barrier()

        @pl.when(sid == 0)
        def _flush():
            pltpu.sync_copy(den_sh, den_out.at[cid])

    return k(e_src, e_dst, a2, zer1)


def _sc_aggregate2(e_src, e_dst, xw2, rd2, ex2, zer16):
    @functools.partial(
        pl.kernel,
        compiler_params=pltpu.CompilerParams(use_tc_tiling_on_sc=False, needs_layout_passes=False),
        out_type=jax.ShapeDtypeStruct((NC, N, L), _f32),
        mesh=_mesh(),
        scratch_types=[
            pltpu.VMEM((N, C2), _f32), pltpu.VMEM((N,), _f32),
            pltpu.VMEM((B,), jnp.int32), pltpu.VMEM((B,), jnp.int32),
            pltpu.VMEM((B,), _f32), pltpu.VMEM((B, L), _f32),
            pltpu.VMEM_SHARED((N, L), _f32),
        ])
    def k(eis, eid, xw2h, rd2h, ex2h, zr, out_p,
          xw2_v, rd2_v, src_v, dst_v, exb, msgb, out_sh):
        cid = lax.axis_index("c")
        sid = lax.axis_index("s")
        wid = cid * NS + sid
        lane_m4 = lax.iota(jnp.int32, L) % 4
        lane0 = lax.iota(jnp.int32, L) * 0

        pltpu.sync_copy(xw2h, xw2_v)
        pltpu.sync_copy(rd2h, rd2_v)

        @pl.when(sid == 0)
        def _zero():
            pltpu.sync_copy(zr, out_sh)
        plsc.subcore_barrier()

        @pl.loop(0, NCH)
        def _chunk(i):
            base = wid * EW + i * B
            pltpu.sync_copy(eis.at[pl.ds(base, B)], src_v)
            pltpu.sync_copy(eid.at[pl.ds(base, B)], dst_v)
            pltpu.sync_copy(ex2h.at[pl.ds(base, B)], exb)

            @pl.loop(0, B // L)
            def _grp(g):
                sv = src_v[pl.ds(g * L, L)]
                dv = dst_v[pl.ds(g * L, L)]
                rd = plsc.load_gather(rd2_v, [dv])
                alpha = exb[pl.ds(g * L, L)] * rd
                for t in range(L):
                    srep = _lane_gather(sv, lane0 + t)
                    arep = _lane_gather(alpha, lane0 + t)
                    xwv = plsc.load_gather(xw2_v, [srep, lane_m4])
                    msgb[g * L + t] = xwv * arep

            pltpu.sync_copy(msgb, out_sh.at[dst_v], add=True)

        plsc.subcore_barrier()

        @pl.when(sid == 0)
        def _flush():
            pltpu.sync_copy(out_sh, out_p.at[cid])

    return k(e_src, e_dst, xw2, rd2, ex2, zer16)


def kernel(x, edge_index, W1, att_src1, att_dst1, bias1,
           W2, att_src2, att_dst2, bias2):
    eye1 = jnp.eye(H1, dtype=_f32)
    As1 = (att_src1[:, :, None] * eye1[:, None, :]).reshape(D1, H1)
    Ad1 = (att_dst1[:, :, None] * eye1[:, None, :]).reshape(D1, H1)
    as2 = att_src2.reshape(C2, 1)
    ad2 = att_dst2.reshape(C2, 1)

    e_src = edge_index[0]
    e_dst = edge_index[1]

    xw1, a1 = _tc_proj(x, W1, As1, Ad1)

    zer16 = jnp.zeros((N, L), _f32)
    ex1, denp1 = _sc_edge_softmax1(e_src, e_dst, a1, zer16)
    rden1 = _tc_rdenom(denp1)

    zer64 = jnp.zeros((N, D1), _f32)
    outp1 = _sc_aggregate1(e_src, e_dst, xw1, rden1, ex1, zer64)

    h, xw2, a2 = _tc_layer2(outp1, bias1.reshape(1, D1), W2, as2, ad2)

    zer1 = jnp.zeros((N,), _f32)
    ex2, denp2 = _sc_edge_softmax2(e_src, e_dst, a2, zer1)
    rd2 = _tc_rdenom(denp2)

    outp2 = _sc_aggregate2(e_src, e_dst, xw2, rd2, ex2, zer16)

    z = _tc_final(outp2, bias2.reshape(1, C2))
    return (h, z)


# speedup vs baseline: 64.4809x; 1.5959x over previous
import functools

import jax
import jax.numpy as jnp
from jax import lax
from jax.experimental import pallas as pl
from jax.experimental.pallas import tpu as pltpu
from jax.experimental.pallas import tpu_sc as plsc

N = 10000
E = 320000
F = 128
H1, C1 = 8, 8
D1 = H1 * C1
C2 = 4
L = 16

NC, NS = 2, 16
NW = NC * NS
EW = E // NW
CH = 80
KJ = 5
B = CH * KJ
NCH = EW // B
RW = EW // CH

_f32 = jnp.float32

_SC_PARAMS = pltpu.CompilerParams(use_tc_tiling_on_sc=False,
                                  needs_layout_passes=False)


def _lane_gather(x, idx):
    return lax.gather(
        x, idx[:, None],
        lax.GatherDimensionNumbers(offset_dims=(), collapsed_slice_dims=(0,),
                                   start_index_map=(0,)),
        (1,), mode=lax.GatherScatterMode.PROMISE_IN_BOUNDS)


def _mesh():
    return plsc.VectorSubcoreMesh(core_axis_name="c", subcore_axis_name="s",
                                  num_cores=NC, num_subcores=NS)


def _tc_proj_body(x_ref, w_ref, as_ref, ad_ref, xw_ref, a1_ref):
    xw = jnp.dot(x_ref[...], w_ref[...], preferred_element_type=_f32)
    xw_ref[...] = xw
    asrc = jnp.dot(xw, as_ref[...], preferred_element_type=_f32)
    adst = jnp.dot(xw, ad_ref[...], preferred_element_type=_f32)
    a1_ref[...] = jnp.concatenate([asrc, adst], axis=1)


def _tc_proj(x, W1, As, Ad):
    bn = 1000
    return pl.pallas_call(
        _tc_proj_body,
        grid=(N // bn,),
        in_specs=[pl.BlockSpec((bn, F), lambda i: (i, 0)),
                  pl.BlockSpec((F, D1), lambda i: (0, 0)),
                  pl.BlockSpec((D1, H1), lambda i: (0, 0)),
                  pl.BlockSpec((D1, H1), lambda i: (0, 0))],
        out_specs=[pl.BlockSpec((bn, D1), lambda i: (i, 0)),
                   pl.BlockSpec((bn, L), lambda i: (i, 0))],
        out_shape=[jax.ShapeDtypeStruct((N, D1), _f32),
                   jax.ShapeDtypeStruct((N, L), _f32)],
    )(x, W1, As, Ad)


def _tc_rdenom_body(d_ref, out_ref):
    out_ref[...] = 1.0 / (d_ref[0] + d_ref[1] + 1e-16)


def _tc_rdenom(denp):
    shp = denp.shape[1:]
    return pl.pallas_call(
        _tc_rdenom_body,
        out_shape=jax.ShapeDtypeStruct(shp, _f32),
    )(denp)


def _tc_layer2_body(op_ref, b1_ref, w2_ref, as2_ref, ad2_ref,
                    h_ref, xw2_ref, a2_ref):
    o = op_ref[0] + op_ref[1] + b1_ref[...]
    h = jnp.maximum(o, 0.0)
    h_ref[...] = h
    xw2 = jnp.dot(h, w2_ref[...], preferred_element_type=_f32)
    xw2_ref[...] = xw2
    asrc2 = jnp.dot(xw2, as2_ref[...], preferred_element_type=_f32)
    adst2 = jnp.dot(xw2, ad2_ref[...], preferred_element_type=_f32)
    a2_ref[...] = jnp.concatenate([asrc2, adst2], axis=1)


def _tc_layer2(outp, bias1, W2, as2, ad2):
    bn = 1000
    return pl.pallas_call(
        _tc_layer2_body,
        grid=(N // bn,),
        in_specs=[pl.BlockSpec((2, bn, D1), lambda i: (0, i, 0)),
                  pl.BlockSpec((1, D1), lambda i: (0, 0)),
                  pl.BlockSpec((D1, C2), lambda i: (0, 0)),
                  pl.BlockSpec((C2, 1), lambda i: (0, 0)),
                  pl.BlockSpec((C2, 1), lambda i: (0, 0))],
        out_specs=[pl.BlockSpec((bn, D1), lambda i: (i, 0)),
                   pl.BlockSpec((bn, C2), lambda i: (i, 0)),
                   pl.BlockSpec((bn, 2), lambda i: (i, 0))],
        out_shape=[jax.ShapeDtypeStruct((N, D1), _f32),
                   jax.ShapeDtypeStruct((N, C2), _f32),
                   jax.ShapeDtypeStruct((N, 2), _f32)],
    )(outp, bias1, W2, as2, ad2)


def _tc_final_body(op_ref, b2_ref, z_ref):
    z_ref[...] = op_ref[0, :, :C2] + op_ref[1, :, :C2] + b2_ref[...]


def _tc_final(out2p, bias2):
    return pl.pallas_call(
        _tc_final_body,
        out_shape=jax.ShapeDtypeStruct((N, C2), _f32),
    )(out2p, bias2)


def _sc_edge_softmax1(e_src, e_dst, a1, zer16):
    @functools.partial(
        pl.kernel,
        compiler_params=_SC_PARAMS,
        out_type=(jax.ShapeDtypeStruct((E, L), _f32),
                  jax.ShapeDtypeStruct((NC, N, L), _f32)),
        mesh=_mesh(),
        scratch_types=[
            pltpu.VMEM((KJ, CH), jnp.int32), pltpu.VMEM((KJ, CH), jnp.int32),
            pltpu.VMEM((B, L), _f32), pltpu.VMEM((B, L), _f32),
            pltpu.VMEM((B, L), _f32),
            pltpu.VMEM_SHARED((N, L), _f32),
            pltpu.SemaphoreType.DMA,
        ])
    def k(eis, eid, a1h, zr, ex_out, den_out,
          src_i, dst_i, srows, drows, exb, den_sh, sem1):
        cid = lax.axis_index("c")
        sid = lax.axis_index("s")
        wid = cid * NS + sid
        rot8 = (lax.iota(jnp.int32, L) % 8) + 8

        @pl.when(sid == 0)
        def _zero():
            pltpu.sync_copy(zr, den_sh)
        plsc.subcore_barrier()

        @pl.loop(0, NCH)
        def _chunk(i):
            base = wid * EW + i * B
            rowb = wid * RW + i * KJ
            pltpu.sync_copy(eis.at[pl.ds(rowb, KJ)], src_i)
            pltpu.sync_copy(eid.at[pl.ds(rowb, KJ)], dst_i)
            descs = []
            for j in range(KJ):
                descs.append(pltpu.async_copy(
                    a1h.at[src_i.at[j]], srows.at[pl.ds(j * CH, CH)], sem1))
                descs.append(pltpu.async_copy(
                    a1h.at[dst_i.at[j]], drows.at[pl.ds(j * CH, CH)], sem1))
            for d in descs:
                d.wait()

            @pl.loop(0, B)
            def _edge(j):
                e = srows[j] + _lane_gather(drows[j], rot8)
                e = jnp.where(e >= 0.0, e, e * _f32(0.2))
                exb[j] = jnp.exp(e)

            pltpu.sync_copy(exb, ex_out.at[pl.ds(base, B)])
            for j in range(KJ):
                pltpu.sync_copy(exb.at[pl.ds(j * CH, CH)],
                                den_sh.at[dst_i.at[j]], add=True)

        plsc.subcore_barrier()

        @pl.when(sid == 0)
        def _flush():
            pltpu.sync_copy(den_sh, den_out.at[cid])

    return k(e_src, e_dst, a1, zer16)


def _sc_aggregate1(e_src, e_dst, xw, rdenom, ex, zer64):
    @functools.partial(
        pl.kernel,
        compiler_params=_SC_PARAMS,
        out_type=jax.ShapeDtypeStruct((NC, N, D1), _f32),
        mesh=_mesh(),
        scratch_types=[
            pltpu.VMEM((KJ, CH), jnp.int32), pltpu.VMEM((KJ, CH), jnp.int32),
            pltpu.VMEM((B, D1), _f32), pltpu.VMEM((B, L), _f32),
            pltpu.VMEM((B, L), _f32), pltpu.VMEM((B, D1), _f32),
            pltpu.VMEM_SHARED((N, D1), _f32),
            pltpu.SemaphoreType.DMA,
        ])
    def k(eis, eid, xwh, rdh, exh, zr, out_p,
          src_i, dst_i, xwrows, rdrows, exb, msgb, out_sh, sem1):
        cid = lax.axis_index("c")
        sid = lax.axis_index("s")
        wid = cid * NS + sid
        lane8 = lax.iota(jnp.int32, L) // 8

        @pl.when(sid == 0)
        def _zero():
            pltpu.sync_copy(zr, out_sh)
        plsc.subcore_barrier()

        @pl.loop(0, NCH)
        def _chunk(i):
            base = wid * EW + i * B
            rowb = wid * RW + i * KJ
            pltpu.sync_copy(eis.at[pl.ds(rowb, KJ)], src_i)
            pltpu.sync_copy(eid.at[pl.ds(rowb, KJ)], dst_i)
            descs = [pltpu.async_copy(exh.at[pl.ds(base, B)], exb, sem1)]
            for j in range(KJ):
                descs.append(pltpu.async_copy(
                    xwh.at[src_i.at[j]], xwrows.at[pl.ds(j * CH, CH)], sem1))
                descs.append(pltpu.async_copy(
                    rdh.at[dst_i.at[j]], rdrows.at[pl.ds(j * CH, CH)], sem1))
            for d in descs:
                d.wait()

            @pl.loop(0, B)
            def _edge(j):
                alpha = exb[j] * rdrows[j]
                for kk in range(4):
                    a_k = _lane_gather(alpha, 2 * kk + lane8)
                    sl = pl.ds(kk * L, L)
                    msgb[j, sl] = xwrows[j, sl] * a_k

            for j in range(KJ):
                pltpu.sync_copy(msgb.at[pl.ds(j * CH, CH)],
                                out_sh.at[dst_i.at[j]], add=True)

        plsc.subcore_barrier()

        @pl.when(sid == 0)
        def _flush():
            pltpu.sync_copy(out_sh, out_p.at[cid])

    return k(e_src, e_dst, xw, rdenom, ex, zer64)


def _sc_edge_softmax2(e_src, e_dst, a2, zer1):
    @functools.partial(
        pl.kernel,
        compiler_params=_SC_PARAMS,
        out_type=(jax.ShapeDtypeStruct((E,), _f32),
                  jax.ShapeDtypeStruct((NC, N), _f32)),
        mesh=_mesh(),
        scratch_types=[
            pltpu.VMEM((N, 2), _f32),
            pltpu.VMEM((KJ, CH), jnp.int32), pltpu.VMEM((KJ, CH), jnp.int32),
            pltpu.VMEM((B,), _f32),
            pltpu.VMEM_SHARED((N,), _f32),
        ])
    def k(eis, eid, a2h, zr, ex_out, den_out,
          a2_v, src_i, dst_i, exb, den_sh):
        cid = lax.axis_index("c")
        sid = lax.axis_index("s")
        wid = cid * NS + sid

        pltpu.sync_copy(a2h, a2_v)

        @pl.when(sid == 0)
        def _zero():
            pltpu.sync_copy(zr, den_sh)
        plsc.subcore_barrier()

        col0 = lax.iota(jnp.int32, L) * 0
        col1 = col0 + 1

        @pl.loop(0, NCH)
        def _chunk(i):
            base = wid * EW + i * B
            rowb = wid * RW + i * KJ
            pltpu.sync_copy(eis.at[pl.ds(rowb, KJ)], src_i)
            pltpu.sync_copy(eid.at[pl.ds(rowb, KJ)], dst_i)

            @pl.loop(0, KJ)
            def _row(r):
                @pl.loop(0, CH // L)
                def _grp(g):
                    sv = src_i[r, pl.ds(g * L, L)]
                    dv = dst_i[r, pl.ds(g * L, L)]
                    av = plsc.load_gather(a2_v, [sv, col0])
                    bv = plsc.load_gather(a2_v, [dv, col1])
                    e = av + bv
                    e = jnp.where(e >= 0.0, e, e * _f32(0.2))
                    exb[pl.ds(r * CH + g * L, L)] = jnp.exp(e)

            pltpu.sync_copy(exb, ex_out.at[pl.ds(base, B)])
            for j in range(KJ):
                pltpu.sync_copy(exb.at[pl.ds(j * CH, CH)],
                                den_sh.at[dst_i.at[j]], add=True)

        plsc.subcore_barrier()

        @pl.when(sid == 0)
        def _flush():
            pltpu.sync_copy(den_sh, den_out.at[cid])

    return k(e_src, e_dst, a2, zer1)


def _sc_aggregate2(e_src, e_dst, xw2, rd2, ex2, zer16):
    @functools.partial(
        pl.kernel,
        compiler_params=_SC_PARAMS,
        out_type=jax.ShapeDtypeStruct((NC, N, L), _f32),
        mesh=_mesh(),
        scratch_types=[
            pltpu.VMEM((N, C2), _f32), pltpu.VMEM((N,), _f32),
            pltpu.VMEM((KJ, CH), jnp.int32), pltpu.VMEM((KJ, CH), jnp.int32),
            pltpu.VMEM((B,), _f32), pltpu.VMEM((B, L), _f32),
            pltpu.VMEM_SHARED((N, L), _f32),
            pltpu.SemaphoreType.DMA,
        ])
    def k(eis, eid, xw2h, rd2h, ex2h, zr, out_p,
          xw2_v, rd2_v, src_i, dst_i, exb, msgb, out_sh, sem1):
        cid = lax.axis_index("c")
        sid = lax.axis_index("s")
        wid = cid * NS + sid
        lane_m4 = lax.iota(jnp.int32, L) % 4
        lane0 = lax.iota(jnp.int32, L) * 0

        pltpu.sync_copy(xw2h, xw2_v)
        pltpu.sync_copy(rd2h, rd2_v)

        @pl.when(sid == 0)
        def _zero():
            pltpu.sync_copy(zr, out_sh)
        plsc.subcore_barrier()

        @pl.loop(0, NCH)
        def _chunk(i):
            base = wid * EW + i * B
            rowb = wid * RW + i * KJ
            pltpu.sync_copy(eis.at[pl.ds(rowb, KJ)], src_i)
            pltpu.sync_copy(eid.at[pl.ds(rowb, KJ)], dst_i)
            pltpu.async_copy(ex2h.at[pl.ds(base, B)], exb, sem1).wait()

            @pl.loop(0, KJ)
            def _row(r):
                @pl.loop(0, CH // L)
                def _grp(g):
                    sv = src_i[r, pl.ds(g * L, L)]
                    dv = dst_i[r, pl.ds(g * L, L)]
                    rd = plsc.load_gather(rd2_v, [dv])
                    alpha = exb[pl.ds(r * CH + g * L, L)] * rd
                    for t in range(L):
                        srep = _lane_gather(sv, lane0 + t)
                        arep = _lane_gather(alpha, lane0 + t)
                        xwv = plsc.load_gather(xw2_v, [srep, lane_m4])
                        msgb[r * CH + g * L + t] = xwv * arep

            for j in range(KJ):
                pltpu.sync_copy(msgb.at[pl.ds(j * CH, CH)],
                                out_sh.at[dst_i.at[j]], add=True)

        plsc.subcore_barrier()

        @pl.when(sid == 0)
        def _flush():
            pltpu.sync_copy(out_sh, out_p.at[cid])

    return k(e_src, e_dst, xw2, rd2, ex2, zer16)


def kernel(x, edge_index, W1, att_src1, att_dst1, bias1,
           W2, att_src2, att_dst2, bias2):
    eye1 = jnp.eye(H1, dtype=_f32)
    As1 = (att_src1[:, :, None] * eye1[:, None, :]).reshape(D1, H1)
    Ad1 = (att_dst1[:, :, None] * eye1[:, None, :]).reshape(D1, H1)
    as2 = att_src2.reshape(C2, 1)
    ad2 = att_dst2.reshape(C2, 1)

    e_src = edge_index[0].reshape(E // CH, CH)
    e_dst = edge_index[1].reshape(E // CH, CH)

    xw1, a1 = _tc_proj(x, W1, As1, Ad1)

    zer16 = jnp.zeros((N, L), _f32)
    ex1, denp1 = _sc_edge_softmax1(e_src, e_dst, a1, zer16)
    rden1 = _tc_rdenom(denp1)

    zer64 = jnp.zeros((N, D1), _f32)
    outp1 = _sc_aggregate1(e_src, e_dst, xw1, rden1, ex1, zer64)

    h, xw2, a2 = _tc_layer2(outp1, bias1.reshape(1, D1), W2, as2, ad2)

    zer1 = jnp.zeros((N,), _f32)
    ex2, denp2 = _sc_edge_softmax2(e_src, e_dst, a2, zer1)
    rd2 = _tc_rdenom(denp2)

    outp2 = _sc_aggregate2(e_src, e_dst, xw2, rd2, ex2, zer16)

    z = _tc_final(outp2, bias2.reshape(1, C2))
    return (h, z)
